# baseline (speedup 1.0000x reference)
import math

import jax
import jax.numpy as jnp
import numpy as np
from jax import lax
from jax.experimental import pallas as pl
from jax.experimental.pallas import tpu as pltpu

N = 65536
R = 8
B = 16
LANES = 128
PHS_ROWS = N // LANES
GRID = 4
PB = PHS_ROWS // GRID
CH = N // GRID

REF_OSC = np.float32((2.0 * math.pi * 40.0 * 0.001) % (2.0 * math.pi))
STEP = np.float32((math.pi / 4.0) / (N - 1))
C2PI = np.float32(2.0 * math.pi)
R_F = np.float32(R)
DECAY = np.float32(0.95)

_LO = float(REF_OSC)
_HI = float(REF_OSC) + math.pi / 4.0
CENTER = np.float32((_LO + _HI) / 2.0)
CC = np.float32(math.cos((_LO + _HI) / 2.0))
SC = np.float32(math.sin((_LO + _HI) / 2.0))

_cols = np.arange(1024)
_src = (_cols // 128) * 16 + ((_cols % 128) // 8)
EXP = (np.arange(128)[:, None] == _src[None, :]).astype(np.float32)

_lin8 = np.linspace(0.0, 2.0 * math.pi, 8)
COS8 = np.cos(_lin8)[:, None].astype(np.float32)
SIN8 = np.sin(_lin8)[:, None].astype(np.float32)


def _body(spkd_ref, spkr_ref, exp_ref, cos8_ref, sin8_ref,
          pe_ref, pbt_ref, phs_ref, lsp_ref, pwt_ref):
    g = pl.program_id(0)

    rows_d = lax.broadcasted_iota(jnp.int32, (PB, LANES), 0)
    lane_d = lax.broadcasted_iota(jnp.int32, (PB, LANES), 1)
    i_d = ((g * PB + rows_d) * LANES + lane_d).astype(jnp.float32)
    phid = REF_OSC + i_d * STEP
    phs_ref[...] = phid
    spkd = spkd_ref[...]
    lsp_ref[...] = jnp.where(spkd > 0, phid, -jnp.inf)

    m1 = lax.dot_general(
        (spkd > 0).astype(jnp.float32), exp_ref[...],
        (((1,), (0,)), ((), ())),
        preferred_element_type=jnp.float32)
    k_f = (lane_d & 7).astype(jnp.float32)
    for r in range(R):
        i_fr = ((g * PB + rows_d) * LANES + np.int32(r * 16)
                + (lane_d >> 3)).astype(jnp.float32)
        phir = REF_OSC + i_fr * STEP
        binr = jnp.floor(phir / C2PI * R_F)
        c_r = DECAY * (binr == k_f).astype(jnp.float32)
        val = m1[:, r * LANES:(r + 1) * LANES] * c_r
        pe_ref[:, :, r, :, :] = jnp.broadcast_to(
            val[None, :, None, :], (2, PB, R, LANES))

    i_t = (g * CH
           + lax.broadcasted_iota(jnp.int32, (R, CH), 1)).astype(jnp.float32)
    k_t = lax.broadcasted_iota(jnp.int32, (R, CH), 0).astype(jnp.float32)
    phit = REF_OSC + i_t * STEP
    bint = jnp.floor(phit / C2PI * R_F)
    maskt = jnp.broadcast_to(spkr_ref[...] > 0, (R, CH))
    pbt_ref[...] = jnp.where(maskt & (bint == k_t), DECAY, np.float32(0.0))
    x = phit - CENTER
    x2 = x * x
    cosx = 1.0 + x2 * (np.float32(-0.5) + x2 * (np.float32(1.0 / 24.0)
                                                + x2 * np.float32(-1.0 / 720.0)))
    sinx = x * (1.0 + x2 * (np.float32(-1.0 / 6.0) + x2 * np.float32(1.0 / 120.0)))
    cphi = CC * cosx - SC * sinx
    sphi = SC * cosx + CC * sinx
    c8 = jnp.broadcast_to(cos8_ref[...], (R, CH))
    s8 = jnp.broadcast_to(sin8_ref[...], (R, CH))
    pwt_ref[...] = cphi * c8 + sphi * s8


def _run(spkd, spkr, expm, cos8, sin8):
    return pl.pallas_call(
        _body,
        grid=(GRID,),
        in_specs=[
            pl.BlockSpec((PB, LANES), lambda g: (g, 0)),
            pl.BlockSpec((1, CH), lambda g: (0, g)),
            pl.BlockSpec((LANES, R * LANES), lambda g: (0, 0)),
            pl.BlockSpec((R, 1), lambda g: (0, 0)),
            pl.BlockSpec((R, 1), lambda g: (0, 0)),
        ],
        out_specs=[
            pl.BlockSpec((2, PB, R, R, LANES), lambda g: (0, g, 0, 0, 0)),
            pl.BlockSpec((R, CH), lambda g: (0, g)),
            pl.BlockSpec((PB, LANES), lambda g: (g, 0)),
            pl.BlockSpec((PB, LANES), lambda g: (g, 0)),
            pl.BlockSpec((R, CH), lambda g: (0, g)),
        ],
        out_shape=[
            jax.ShapeDtypeStruct((2, PHS_ROWS, R, R, LANES), jnp.float32),
            jax.ShapeDtypeStruct((R, N), jnp.float32),
            jax.ShapeDtypeStruct((PHS_ROWS, LANES), jnp.float32),
            jax.ShapeDtypeStruct((PHS_ROWS, LANES), jnp.float32),
            jax.ShapeDtypeStruct((R, N), jnp.float32),
        ],
        compiler_params=pltpu.CompilerParams(
            dimension_semantics=("parallel",)),
    )(spkd, spkr, expm, cos8, sin8)


def kernel(input_spikes, current_time):
    spkd = input_spikes.reshape(B * PHS_ROWS, LANES)
    spkr = input_spikes[0].reshape(1, N)
    pe5, pbt, phs, lsp, pwt = _run(
        spkd, spkr, jnp.asarray(EXP), jnp.asarray(COS8), jnp.asarray(SIN8))
    phase_encoded = jnp.transpose(pe5, (0, 3, 1, 2, 4)).reshape(B, N * R)
    current_phases = phs.reshape(N)
    phase_bins = jnp.transpose(pbt).reshape(N, R)
    reference_phase = jnp.asarray(REF_OSC, dtype=jnp.float32)
    last_spike_phases = lsp.reshape(N)
    phase_weights = jnp.transpose(pwt).reshape(N, R)
    return (phase_encoded, current_phases, phase_bins, reference_phase,
            last_spike_phases, phase_weights)


# speedup vs baseline: 1.2482x; 1.2482x over previous
import math

import jax
import jax.numpy as jnp
import numpy as np
from jax import lax
from jax.experimental import pallas as pl
from jax.experimental.pallas import tpu as pltpu

N = 65536
R = 8
B = 16
LANES = 128
PHS_ROWS = N // LANES
GRID = 4
PB = PHS_ROWS // GRID
CH = N // GRID

REF_OSC = np.float32((2.0 * math.pi * 40.0 * 0.001) % (2.0 * math.pi))
STEP = np.float32((math.pi / 4.0) / (N - 1))
C2PI = np.float32(2.0 * math.pi)
R_F = np.float32(R)
DECAY = np.float32(0.95)

_LO = float(REF_OSC)
_HI = float(REF_OSC) + math.pi / 4.0
CENTER = np.float32((_LO + _HI) / 2.0)
CC = np.float32(math.cos((_LO + _HI) / 2.0))
SC = np.float32(math.sin((_LO + _HI) / 2.0))

_cols = np.arange(1024)
_src = (_cols // 128) * 16 + ((_cols % 128) // 8)
EXP = (np.arange(128)[:, None] == _src[None, :]).astype(np.float32)

_lin8 = np.linspace(0.0, 2.0 * math.pi, 8)
COS8 = np.cos(_lin8)[:, None].astype(np.float32)
SIN8 = np.sin(_lin8)[:, None].astype(np.float32)


def _body(spkd_ref, spkr_ref, exp_ref, cos8_ref, sin8_ref,
          pe_ref, pbt_ref, phs_ref, lsp_ref, pwt_ref):
    g = pl.program_id(0)

    rows_d = lax.broadcasted_iota(jnp.int32, (PB, LANES), 0)
    lane_d = lax.broadcasted_iota(jnp.int32, (PB, LANES), 1)
    i_d = ((g * PB + rows_d) * LANES + lane_d).astype(jnp.float32)
    phid = REF_OSC + i_d * STEP
    phs_ref[...] = phid
    spkd = spkd_ref[...]
    lsp_ref[...] = jnp.where(spkd > 0, phid, -jnp.inf)

    m1 = lax.dot_general(
        (spkd > 0).astype(jnp.float32), exp_ref[...],
        (((1,), (0,)), ((), ())),
        preferred_element_type=jnp.float32)
    k_f = (lane_d & 7).astype(jnp.float32)
    for r in range(R):
        i_fr = ((g * PB + rows_d) * LANES + np.int32(r * 16)
                + (lane_d >> 3)).astype(jnp.float32)
        phir = REF_OSC + i_fr * STEP
        binr = jnp.floor(phir / C2PI * R_F)
        c_r = DECAY * (binr == k_f).astype(jnp.float32)
        val = m1[:, r * LANES:(r + 1) * LANES] * c_r
        pe_ref[:, :, r, :, :] = jnp.broadcast_to(
            val[None, :, None, :], (2, PB, R, LANES))

    i_t = (g * CH
           + lax.broadcasted_iota(jnp.int32, (R, CH), 1)).astype(jnp.float32)
    k_t = lax.broadcasted_iota(jnp.int32, (R, CH), 0).astype(jnp.float32)
    phit = REF_OSC + i_t * STEP
    bint = jnp.floor(phit / C2PI * R_F)
    maskt = jnp.broadcast_to(spkr_ref[...] > 0, (R, CH))
    pbt_ref[...] = jnp.where(maskt & (bint == k_t), DECAY, np.float32(0.0))
    x = phit - CENTER
    x2 = x * x
    cosx = 1.0 + x2 * (np.float32(-0.5) + x2 * (np.float32(1.0 / 24.0)
                                                + x2 * np.float32(-1.0 / 720.0)))
    sinx = x * (1.0 + x2 * (np.float32(-1.0 / 6.0) + x2 * np.float32(1.0 / 120.0)))
    cphi = CC * cosx - SC * sinx
    sphi = SC * cosx + CC * sinx
    c8 = jnp.broadcast_to(cos8_ref[...], (R, CH))
    s8 = jnp.broadcast_to(sin8_ref[...], (R, CH))
    pwt_ref[...] = cphi * c8 + sphi * s8


def _run(spkd, spkr, expm, cos8, sin8):
    return pl.pallas_call(
        _body,
        grid=(GRID,),
        in_specs=[
            pl.BlockSpec((PB, LANES), lambda g: (g, 0)),
            pl.BlockSpec((1, CH), lambda g: (0, g)),
            pl.BlockSpec((LANES, R * LANES), lambda g: (0, 0)),
            pl.BlockSpec((R, 1), lambda g: (0, 0)),
            pl.BlockSpec((R, 1), lambda g: (0, 0)),
        ],
        out_specs=[
            pl.BlockSpec((2, PB, R, R, LANES), lambda g: (0, g, 0, 0, 0)),
            pl.BlockSpec((R, CH), lambda g: (0, g)),
            pl.BlockSpec((PB, LANES), lambda g: (g, 0)),
            pl.BlockSpec((PB, LANES), lambda g: (g, 0)),
            pl.BlockSpec((R, CH), lambda g: (0, g)),
        ],
        out_shape=[
            jax.ShapeDtypeStruct((2, PHS_ROWS, R, R, LANES), jnp.float32),
            jax.ShapeDtypeStruct((R, N), jnp.float32),
            jax.ShapeDtypeStruct((PHS_ROWS, LANES), jnp.float32),
            jax.ShapeDtypeStruct((PHS_ROWS, LANES), jnp.float32),
            jax.ShapeDtypeStruct((R, N), jnp.float32),
        ],
        compiler_params=pltpu.CompilerParams(
            dimension_semantics=("parallel",)),
    )(spkd, spkr, expm, cos8, sin8)


def kernel(input_spikes, current_time):
    row0 = input_spikes[0]
    spkd = row0.reshape(PHS_ROWS, LANES)
    spkr = row0.reshape(1, N)
    pe5, pbt, phs, lsp, pwt = _run(
        spkd, spkr, jnp.asarray(EXP), jnp.asarray(COS8), jnp.asarray(SIN8))
    phase_encoded = jnp.transpose(pe5, (0, 3, 1, 2, 4)).reshape(B, N * R)
    current_phases = phs.reshape(N)
    phase_bins = jnp.transpose(pbt).reshape(N, R)
    reference_phase = jnp.asarray(REF_OSC, dtype=jnp.float32)
    last_spike_phases = lsp.reshape(N)
    phase_weights = jnp.transpose(pwt).reshape(N, R)
    return (phase_encoded, current_phases, phase_bins, reference_phase,
            last_spike_phases, phase_weights)
